# baseline (speedup 1.0000x reference)
import functools

import jax
import jax.numpy as jnp
import numpy as np
from jax.experimental import pallas as pl

_B, _S, _C = 4, 512, 48
_R = _B * _S
_P = _R
_NNEG = 5 * _P
_MARGIN = 0.3
_ALPHA = 0.5
_SMOOTH = 0.1
_BASE = _SMOOTH / (_C - 1)

_INT_MIN = np.int32(-2147483648)
_MASK31 = np.int32(0x7FFFFFFF)
_NEG_SENTINEL = -1.0e9


def _sort_key(x_f32):
    i = jax.lax.bitcast_convert_type(x_f32, jnp.int32)
    return i ^ ((i >> 31) & _MASK31)


def _key_to_float(k_i32):
    return jax.lax.bitcast_convert_type(k_i32 ^ ((k_i32 >> 31) & _MASK31),
                                        jnp.float32)


def _margin_kernel(edge_ref, heads_ref, out_ref):
    edge = edge_ref[...]
    heads = heads_ref[...]

    col = jax.lax.broadcasted_iota(jnp.int32, (_R, _S), 1)
    sel = col == heads
    pos = jnp.sum(jnp.where(sel, edge, 0.0), axis=1, keepdims=True)

    keys = _sort_key(edge)
    pkeys = _sort_key(pos)

    def count_ge(t):
        return (jnp.sum((keys >= t).astype(jnp.int32))
                - jnp.sum((pkeys >= t).astype(jnp.int32)))

    t0 = jnp.where(count_ge(jnp.int32(0)) >= _NNEG, jnp.int32(0), _INT_MIN)

    def bs_body(k, t):
        trial = t + jax.lax.shift_left(jnp.int32(1), jnp.int32(30) - k)
        return jnp.where(count_ge(trial) >= _NNEG, trial, t)

    tau_k = jax.lax.fori_loop(0, 31, bs_body, t0)

    cnt_gt = (jnp.sum((keys > tau_k).astype(jnp.int32))
              - jnp.sum((pkeys > tau_k).astype(jnp.int32)))
    ties = (_NNEG - cnt_gt).astype(jnp.float32)
    tau_f = _key_to_float(tau_k)

    gated = jnp.where(keys > tau_k, edge, _NEG_SENTINEL)
    gated_pos = jnp.where(pkeys > tau_k, pos, _NEG_SENTINEL)

    row = jax.lax.broadcasted_iota(jnp.int32, (_R, 1), 0)

    def pair_body(i, acc):
        p_i = jnp.sum(jnp.where(row == i, pos, 0.0))
        c = _MARGIN - p_i
        s_main = jnp.sum(jnp.maximum(gated + c, 0.0))
        s_pos = jnp.sum(jnp.maximum(gated_pos + c, 0.0))
        s_tie = ties * jnp.maximum(tau_f + c, 0.0)
        return acc + (s_main - s_pos + s_tie)

    total = jax.lax.fori_loop(0, _R, pair_body, jnp.float32(0.0))
    out_ref[...] = jnp.full((1, 1), total / jnp.float32(_P * _NNEG),
                            jnp.float32)


_CHB = 32


def _label_kernel(labels_ref, heads_ref, gold_ref, out_ref):
    i = pl.program_id(0)
    x3 = labels_ref[...]
    h = heads_ref[...]
    tgt = gold_ref[...]

    hsel = jax.lax.broadcasted_iota(jnp.int32, (_CHB, _S, _C), 1) == h[:, :, None]
    g = jnp.sum(jnp.where(hsel, x3, 0.0), axis=1)

    m = jnp.max(g, axis=1, keepdims=True)
    lse = m + jnp.log(jnp.sum(jnp.exp(g - m), axis=1, keepdims=True))
    sum_logp = jnp.sum(g, axis=1, keepdims=True) - _C * lse

    lane = jax.lax.broadcasted_iota(jnp.int32, (_CHB, _C), 1)
    xt = jnp.sum(jnp.where(lane == tgt, g, 0.0), axis=1, keepdims=True)
    logp_t = xt - lse

    row_loss = -(_BASE * sum_logp) - (1.0 - _SMOOTH - _BASE) * logp_t
    partial = jnp.sum(row_loss)
    prev = jnp.where(i == 0, jnp.zeros((1, 1), jnp.float32), out_ref[...])
    out_ref[...] = prev + jnp.full((1, 1), partial, jnp.float32)


@jax.jit
def kernel(edge_scores, label_scores, gold_heads, gold_labels):
    edge2d = edge_scores.reshape(_R, _S)
    heads2d = gold_heads.reshape(_R, 1)
    gold2d = gold_labels.reshape(_R, 1)
    labels3d = label_scores.reshape(_R, _S, _C)

    margin = pl.pallas_call(
        _margin_kernel,
        out_shape=jax.ShapeDtypeStruct((1, 1), jnp.float32),
    )(edge2d, heads2d)

    label = pl.pallas_call(
        _label_kernel,
        grid=(_R // _CHB,),
        in_specs=[
            pl.BlockSpec((_CHB, _S, _C), lambda i: (i, 0, 0)),
            pl.BlockSpec((_CHB, 1), lambda i: (i, 0)),
            pl.BlockSpec((_CHB, 1), lambda i: (i, 0)),
        ],
        out_specs=pl.BlockSpec((1, 1), lambda i: (0, 0)),
        out_shape=jax.ShapeDtypeStruct((1, 1), jnp.float32),
    )(labels3d, heads2d, gold2d)

    margin_loss = margin[0, 0]
    label_loss = label[0, 0] / jnp.float32(_R)
    return _ALPHA * margin_loss + (1.0 - _ALPHA) * label_loss


# speedup vs baseline: 5.4084x; 5.4084x over previous
import functools

import jax
import jax.numpy as jnp
import numpy as np
from jax import lax
from jax.experimental import pallas as pl
from jax.experimental.pallas import tpu as pltpu
from jax.experimental.pallas import tpu_sc as plsc

_B, _S, _C = 4, 512, 48
_R = _B * _S
_P = _R
_NNEG = 5 * _P
_MARGIN = 0.3
_ALPHA = 0.5
_SMOOTH = 0.1
_BASE = _SMOOTH / (_C - 1)

_NW = 32
_GRP = _R // _NW
_CHUNK = _GRP * _S
_CAP = 12288
_PAD = 256
_TOTCAP = _CAP + _NW * _PAD
_ROWCAP = _CAP // _PAD

_INT_MIN = np.int32(-2147483648)
_NEG_SENTINEL = -1.0e9


def _sort_key(i):
    return i ^ (lax.shift_right_arithmetic(i, 31) & 0x7FFFFFFF)


def _f32_key(x):
    return _sort_key(lax.bitcast_convert_type(x, jnp.int32))


def _stats_kernel(edge_ref, heads_ref, pos_ref, rowidx_ref, taub_ref,
                  tstats_ref):
    edge = edge_ref[...]
    heads = heads_ref[...]

    col = lax.broadcasted_iota(jnp.int32, (_NW, _GRP, _S), 2)
    sel = col == heads[:, :, None]
    pos = jnp.sum(jnp.where(sel, edge, 0.0), axis=2)

    keys = _f32_key(edge)
    pkeys = _f32_key(pos)

    def count_ge(t):
        return (jnp.sum((keys >= t).astype(jnp.int32))
                - jnp.sum((pkeys >= t).astype(jnp.int32)))

    t0 = jnp.where(count_ge(jnp.int32(0)) >= _NNEG, jnp.int32(0), _INT_MIN)

    def bs_body(k, t):
        trial = t + lax.shift_left(jnp.int32(1), jnp.int32(30) - k)
        return jnp.where(count_ge(trial) >= _NNEG, trial, t)

    tau = lax.fori_loop(0, 31, bs_body, t0)

    cnt_gt = (jnp.sum((keys > tau).astype(jnp.int32))
              - jnp.sum((pkeys > tau).astype(jnp.int32)))

    gt = (keys > tau).astype(jnp.int32)
    counts = jnp.sum(jnp.sum(gt, axis=2), axis=1, keepdims=True)
    rc = (counts + (_PAD - 1)) & (-_PAD)
    total = jnp.sum(rc)
    row = lax.broadcasted_iota(jnp.int32, (_NW, 1), 0)

    def off_body(wi, acc):
        pre = jnp.sum(jnp.where(row < wi, rc, 0))
        return jnp.where(row == wi, pre, acc)

    offs = lax.fori_loop(0, _NW, off_body, jnp.zeros((_NW, 1), jnp.int32))

    pos_ref[...] = pos
    colr = lax.broadcasted_iota(jnp.int32, (_NW, _ROWCAP), 1)
    rowidx_ref[...] = jnp.where(
        colr < lax.shift_right_logical(rc, 8),
        lax.shift_right_logical(offs, 8) + colr, -1)
    taub_ref[...] = jnp.full((1, 16), 0, jnp.int32) + tau
    lane = lax.broadcasted_iota(jnp.int32, (1, 128), 1)
    tstats_ref[...] = jnp.where(
        lane == 0, tau,
        jnp.where(lane == 1, cnt_gt, jnp.where(lane == 2, total, 0)))


def _sc_filter_body(edge_ref, tau_ref, staging_ref, vchunk, cbuf, tbuf):
    w = lax.axis_index("s") * 2 + lax.axis_index("c")

    pltpu.sync_copy(tau_ref, tbuf)
    tau_v = tbuf[...]

    pltpu.sync_copy(edge_ref.at[pl.ds(w * _CHUNK, _CHUNK)], vchunk)

    def fill_body(j, _):
        cbuf[pl.ds(j * 16, 16)] = jnp.full((16,), _NEG_SENTINEL, jnp.float32)
        return 0

    lax.fori_loop(0, _CAP // 16, fill_body, 0)

    def filt_body(j, wp):
        v = vchunk[pl.ds(j * 16, 16)]
        key = _sort_key(plsc.bitcast(v, jnp.int32))
        m = key > tau_v
        mi = m.astype(jnp.int32)
        dst = wp + plsc.cumsum(mi) - 1
        plsc.store_scatter(cbuf, [dst], v, mask=m)
        return wp + jnp.sum(mi)

    lax.fori_loop(0, _CHUNK // 16, filt_body, jnp.int32(0))

    pltpu.sync_copy(cbuf, staging_ref.at[pl.ds(w * _CAP, _CAP)])


def _sc_pack_body(staging_ref, rowidx_ref, labels_ref, heads_ref,
                  compact_ref, gathered_ref,
                  sbuf, ridx, hbuf, idxbuf, lrows, sem, sem2):
    w = lax.axis_index("s") * 2 + lax.axis_index("c")
    iota = lax.iota(jnp.int32, 16)

    pltpu.sync_copy(staging_ref.at[pl.ds(w * _ROWCAP, _ROWCAP)], sbuf)
    pltpu.sync_copy(rowidx_ref.at[pl.ds(w * _ROWCAP, _ROWCAP)], ridx)
    pltpu.async_copy(
        sbuf, compact_ref.at[plsc.Indices(ridx, ignored_value=-1)],
        sem2).wait()

    rbase = w * _GRP
    pltpu.sync_copy(heads_ref.at[pl.ds(rbase, _GRP)], hbuf.at[pl.ds(0, _GRP)])
    for k in range(_GRP // 16):
        h = hbuf[pl.ds(k * 16, 16)]
        idxbuf[pl.ds(k * 16, 16)] = (rbase + k * 16 + iota) * _S + h
    pltpu.async_copy(labels_ref.at[idxbuf], lrows, sem).wait()
    pltpu.sync_copy(lrows, gathered_ref.at[pl.ds(rbase, _GRP)])


def _sc_mesh():
    return plsc.VectorSubcoreMesh(core_axis_name="c", subcore_axis_name="s",
                                  num_cores=2, num_subcores=16)


@functools.cache
def _sc_filter():
    return pl.kernel(
        _sc_filter_body,
        out_type=jax.ShapeDtypeStruct((_NW * _CAP,), jnp.float32),
        mesh=_sc_mesh(),
        compiler_params=pltpu.CompilerParams(use_tc_tiling_on_sc=False,
                                             needs_layout_passes=False),
        scratch_types=[
            pltpu.VMEM((_CHUNK,), jnp.float32),
            pltpu.VMEM((_CAP,), jnp.float32),
            pltpu.VMEM((16,), jnp.int32),
        ],
    )


@functools.cache
def _sc_pack():
    return pl.kernel(
        _sc_pack_body,
        out_type=(
            jax.ShapeDtypeStruct((_TOTCAP // _PAD, _PAD), jnp.float32),
            jax.ShapeDtypeStruct((_R, _C), jnp.float32),
        ),
        mesh=_sc_mesh(),
        compiler_params=pltpu.CompilerParams(use_tc_tiling_on_sc=False),
        scratch_types=[
            pltpu.VMEM((_ROWCAP, _PAD), jnp.float32),
            pltpu.VMEM((_ROWCAP,), jnp.int32),
            pltpu.VMEM((_GRP,), jnp.int32),
            pltpu.VMEM((_GRP,), jnp.int32),
            pltpu.VMEM((_GRP, _C), jnp.float32),
            pltpu.SemaphoreType.DMA,
            pltpu.SemaphoreType.DMA,
        ],
    )


def _final_kernel(crow_ref, pos_col_ref, pos_row_ref, tstats_ref,
                  gathered_ref, gold_ref, out_ref):
    lane = lax.broadcasted_iota(jnp.int32, (1, 128), 1)
    tstats = tstats_ref[...]
    tau = jnp.sum(jnp.where(lane == 0, tstats, 0))
    cnt_gt = jnp.sum(jnp.where(lane == 1, tstats, 0))
    total = jnp.sum(jnp.where(lane == 2, tstats, 0))
    ties = (_NNEG - cnt_gt).astype(jnp.float32)
    tau_f = lax.bitcast_convert_type(_sort_key(tau), jnp.float32)

    pos_col = pos_col_ref[...]
    pos_row = pos_row_ref[...]

    def pair_body(k, acc):
        c = crow_ref[:, pl.ds(k * 128, 128)]
        gidx = k * 128 + lax.broadcasted_iota(jnp.int32, (1, 128), 1)
        c = jnp.where(gidx < total, c, _NEG_SENTINEL)
        return acc + jnp.sum(jnp.maximum(c + (_MARGIN - pos_col), 0.0))

    acc = lax.fori_loop(0, _TOTCAP // 128, pair_body, jnp.float32(0.0))

    gp_row = jnp.where(_f32_key(pos_row) > tau, pos_row, _NEG_SENTINEL)
    acc = acc - jnp.sum(jnp.maximum(gp_row + (_MARGIN - pos_col), 0.0))
    acc = acc + ties * jnp.sum(jnp.maximum(tau_f + (_MARGIN - pos_col), 0.0))

    margin_loss = acc / jnp.float32(_P * _NNEG)

    g = gathered_ref[...]
    tgt = gold_ref[...]
    m = jnp.max(g, axis=1, keepdims=True)
    lse = m + jnp.log(jnp.sum(jnp.exp(g - m), axis=1, keepdims=True))
    sum_logp = jnp.sum(g, axis=1, keepdims=True) - _C * lse
    clane = lax.broadcasted_iota(jnp.int32, (_R, _C), 1)
    xt = jnp.sum(jnp.where(clane == tgt, g, 0.0), axis=1, keepdims=True)
    logp_t = xt - lse
    row_loss = -(_BASE * sum_logp) - (1.0 - _SMOOTH - _BASE) * logp_t
    label_loss = jnp.sum(row_loss) / jnp.float32(_R)

    out_ref[...] = jnp.full((1, 1),
                            _ALPHA * margin_loss + (1.0 - _ALPHA) * label_loss,
                            jnp.float32)


@jax.jit
def kernel(edge_scores, label_scores, gold_heads, gold_labels):
    edge3 = edge_scores.reshape(_NW, _GRP, _S)
    heads2 = gold_heads.reshape(_NW, _GRP)
    heads_flat = gold_heads.reshape(_R)
    gold2 = gold_labels.reshape(_R, 1)
    labels2 = label_scores.reshape(_R * _S, _C)
    edge_flat = edge_scores.reshape(_R * _S)

    pos, rowidx, taub, tstats = pl.pallas_call(
        _stats_kernel,
        out_shape=(
            jax.ShapeDtypeStruct((_NW, _GRP), jnp.float32),
            jax.ShapeDtypeStruct((_NW, _ROWCAP), jnp.int32),
            jax.ShapeDtypeStruct((1, 16), jnp.int32),
            jax.ShapeDtypeStruct((1, 128), jnp.int32),
        ),
    )(edge3, heads2)

    staging = _sc_filter()(edge_flat, taub.reshape(16))
    compact, gathered = _sc_pack()(staging.reshape(_NW * _ROWCAP, _PAD),
                                   rowidx.reshape(_NW * _ROWCAP),
                                   labels2, heads_flat)

    out = pl.pallas_call(
        _final_kernel,
        out_shape=jax.ShapeDtypeStruct((1, 1), jnp.float32),
    )(compact.reshape(1, _TOTCAP), pos.reshape(_P, 1), pos.reshape(1, _P),
      tstats, gathered, gold2)

    return out[0, 0]


# speedup vs baseline: 14.7289x; 2.7233x over previous
import functools

import jax
import jax.numpy as jnp
import numpy as np
from jax import lax
from jax.experimental import pallas as pl
from jax.experimental.pallas import tpu as pltpu
from jax.experimental.pallas import tpu_sc as plsc

_B, _S, _C = 4, 512, 48
_R = _B * _S
_P = _R
_NNEG = 5 * _P
_MARGIN = 0.3
_ALPHA = 0.5
_SMOOTH = 0.1
_BASE = _SMOOTH / (_C - 1)

_NW = 32
_GRP = _R // _NW
_CHUNK = _GRP * _S
_CAP = 12288
_PAD = 256
_TOTCAP = _CAP + _NW * _PAD
_ROWCAP = _CAP // _PAD

_INT_MIN = np.int32(-2147483648)
_NEG_SENTINEL = -1.0e9


def _sort_key(i):
    return i ^ (lax.shift_right_arithmetic(i, 31) & 0x7FFFFFFF)


def _f32_key(x):
    return _sort_key(lax.bitcast_convert_type(x, jnp.int32))


def _stats_kernel(edge_ref, heads_ref, pos_ref, rowidx_ref, taub_ref,
                  tstats_ref):
    edge = edge_ref[...]
    heads = heads_ref[...]

    col = lax.broadcasted_iota(jnp.int32, (_NW, _GRP, _S), 2)
    sel = col == heads[:, :, None]
    pos = jnp.sum(jnp.where(sel, edge, 0.0), axis=2)

    keys = _f32_key(edge)
    pkeys = _f32_key(pos)

    def count_ge(t):
        return (jnp.sum((keys >= t).astype(jnp.int32))
                - jnp.sum((pkeys >= t).astype(jnp.int32)))

    t0 = jnp.where(count_ge(jnp.int32(0)) >= _NNEG, jnp.int32(0), _INT_MIN)

    def bs_body(k, t):
        trial = t + lax.shift_left(jnp.int32(1), jnp.int32(30) - k)
        return jnp.where(count_ge(trial) >= _NNEG, trial, t)

    tau = lax.fori_loop(0, 31, bs_body, t0)

    cnt_gt = (jnp.sum((keys > tau).astype(jnp.int32))
              - jnp.sum((pkeys > tau).astype(jnp.int32)))

    gt = (keys > tau).astype(jnp.int32)
    counts = jnp.sum(jnp.sum(gt, axis=2), axis=1, keepdims=True)
    rc = (counts + (_PAD - 1)) & (-_PAD)
    total = jnp.sum(rc)
    row = lax.broadcasted_iota(jnp.int32, (_NW, 1), 0)

    def off_body(wi, acc):
        pre = jnp.sum(jnp.where(row < wi, rc, 0))
        return jnp.where(row == wi, pre, acc)

    offs = lax.fori_loop(0, _NW, off_body, jnp.zeros((_NW, 1), jnp.int32))

    pos_ref[...] = pos
    colr = lax.broadcasted_iota(jnp.int32, (_NW, _ROWCAP), 1)
    rowidx_ref[...] = jnp.where(
        colr < lax.shift_right_logical(rc, 8),
        lax.shift_right_logical(offs, 8) + colr, -1)
    taub_ref[...] = jnp.full((1, 16), 0, jnp.int32) + tau
    lane = lax.broadcasted_iota(jnp.int32, (1, 128), 1)
    tstats_ref[...] = jnp.where(
        lane == 0, tau,
        jnp.where(lane == 1, cnt_gt, jnp.where(lane == 2, total, 0)))


def _sc_filter_body(edge_ref, tau_ref, staging_ref, vchunk, cbuf, tbuf):
    w = lax.axis_index("s") * 2 + lax.axis_index("c")

    pltpu.sync_copy(tau_ref, tbuf)
    tau_v = tbuf[...]

    pltpu.sync_copy(edge_ref.at[pl.ds(w * _CHUNK, _CHUNK)], vchunk)

    def fill_body(j, _):
        cbuf[pl.ds(j * 16, 16)] = jnp.full((16,), _NEG_SENTINEL, jnp.float32)
        return 0

    lax.fori_loop(0, _CAP // 16, fill_body, 0)

    def filt_body(j, wp):
        v = vchunk[pl.ds(j * 16, 16)]
        key = _sort_key(plsc.bitcast(v, jnp.int32))
        m = key > tau_v
        mi = m.astype(jnp.int32)
        dst = wp + plsc.cumsum(mi) - 1
        plsc.store_scatter(cbuf, [dst], v, mask=m)
        return wp + jnp.sum(mi)

    lax.fori_loop(0, _CHUNK // 16, filt_body, jnp.int32(0))

    pltpu.sync_copy(cbuf, staging_ref.at[pl.ds(w * _CAP, _CAP)])


def _sc_pack_body(staging_ref, rowidx_ref, compact_ref, sbuf, ridx, sem2):
    w = lax.axis_index("s") * 2 + lax.axis_index("c")

    pltpu.sync_copy(staging_ref.at[pl.ds(w * _ROWCAP, _ROWCAP)], sbuf)
    pltpu.sync_copy(rowidx_ref.at[pl.ds(w * _ROWCAP, _ROWCAP)], ridx)
    pltpu.async_copy(
        sbuf, compact_ref.at[plsc.Indices(ridx, ignored_value=-1)],
        sem2).wait()


def _sc_mesh():
    return plsc.VectorSubcoreMesh(core_axis_name="c", subcore_axis_name="s",
                                  num_cores=2, num_subcores=16)


@functools.cache
def _sc_filter():
    return pl.kernel(
        _sc_filter_body,
        out_type=jax.ShapeDtypeStruct((_NW * _CAP,), jnp.float32),
        mesh=_sc_mesh(),
        compiler_params=pltpu.CompilerParams(use_tc_tiling_on_sc=False,
                                             needs_layout_passes=False),
        scratch_types=[
            pltpu.VMEM((_CHUNK,), jnp.float32),
            pltpu.VMEM((_CAP,), jnp.float32),
            pltpu.VMEM((16,), jnp.int32),
        ],
    )


@functools.cache
def _sc_pack():
    return pl.kernel(
        _sc_pack_body,
        out_type=jax.ShapeDtypeStruct((_TOTCAP // _PAD, _PAD), jnp.float32),
        mesh=_sc_mesh(),
        compiler_params=pltpu.CompilerParams(use_tc_tiling_on_sc=False),
        scratch_types=[
            pltpu.VMEM((_ROWCAP, _PAD), jnp.float32),
            pltpu.VMEM((_ROWCAP,), jnp.int32),
            pltpu.SemaphoreType.DMA,
        ],
    )


_CHL = 32


def _label_kernel(labels_ref, heads_ref, gathered_ref):
    x3 = labels_ref[...]
    h = heads_ref[...]
    hsel = lax.broadcasted_iota(jnp.int32, (_CHL, _C, _S), 2) == h[:, :, None]
    gathered_ref[...] = jnp.sum(jnp.where(hsel, x3, 0.0), axis=2)


def _final_kernel(crow_ref, pos_col_ref, pos_row_ref, tstats_ref,
                  gathered_ref, gold_ref, out_ref):
    lane = lax.broadcasted_iota(jnp.int32, (1, 128), 1)
    tstats = tstats_ref[...]
    tau = jnp.sum(jnp.where(lane == 0, tstats, 0))
    cnt_gt = jnp.sum(jnp.where(lane == 1, tstats, 0))
    total = jnp.sum(jnp.where(lane == 2, tstats, 0))
    ties = (_NNEG - cnt_gt).astype(jnp.float32)
    tau_f = lax.bitcast_convert_type(_sort_key(tau), jnp.float32)

    pos_col = pos_col_ref[...]
    pos_row = pos_row_ref[...]

    def pair_body(k, acc):
        c = crow_ref[:, pl.ds(k * 128, 128)]
        gidx = k * 128 + lax.broadcasted_iota(jnp.int32, (1, 128), 1)
        c = jnp.where(gidx < total, c, _NEG_SENTINEL)
        return acc + jnp.sum(jnp.maximum(c + (_MARGIN - pos_col), 0.0))

    acc = lax.fori_loop(0, _TOTCAP // 128, pair_body, jnp.float32(0.0))

    gp_row = jnp.where(_f32_key(pos_row) > tau, pos_row, _NEG_SENTINEL)
    acc = acc - jnp.sum(jnp.maximum(gp_row + (_MARGIN - pos_col), 0.0))
    acc = acc + ties * jnp.sum(jnp.maximum(tau_f + (_MARGIN - pos_col), 0.0))

    margin_loss = acc / jnp.float32(_P * _NNEG)

    g = gathered_ref[...]
    tgt = gold_ref[...]
    m = jnp.max(g, axis=1, keepdims=True)
    lse = m + jnp.log(jnp.sum(jnp.exp(g - m), axis=1, keepdims=True))
    sum_logp = jnp.sum(g, axis=1, keepdims=True) - _C * lse
    clane = lax.broadcasted_iota(jnp.int32, (_R, _C), 1)
    xt = jnp.sum(jnp.where(clane == tgt, g, 0.0), axis=1, keepdims=True)
    logp_t = xt - lse
    row_loss = -(_BASE * sum_logp) - (1.0 - _SMOOTH - _BASE) * logp_t
    label_loss = jnp.sum(row_loss) / jnp.float32(_R)

    out_ref[...] = jnp.full((1, 1),
                            _ALPHA * margin_loss + (1.0 - _ALPHA) * label_loss,
                            jnp.float32)


@jax.jit
def kernel(edge_scores, label_scores, gold_heads, gold_labels):
    edge3 = edge_scores.reshape(_NW, _GRP, _S)
    heads2 = gold_heads.reshape(_NW, _GRP)
    gold2 = gold_labels.reshape(_R, 1)
    labels_t = label_scores.transpose(0, 1, 3, 2).reshape(_R, _C, _S)
    edge_flat = edge_scores.reshape(_R * _S)

    pos, rowidx, taub, tstats = pl.pallas_call(
        _stats_kernel,
        out_shape=(
            jax.ShapeDtypeStruct((_NW, _GRP), jnp.float32),
            jax.ShapeDtypeStruct((_NW, _ROWCAP), jnp.int32),
            jax.ShapeDtypeStruct((1, 16), jnp.int32),
            jax.ShapeDtypeStruct((1, 128), jnp.int32),
        ),
    )(edge3, heads2)

    staging = _sc_filter()(edge_flat, taub.reshape(16))
    compact = _sc_pack()(staging.reshape(_NW * _ROWCAP, _PAD),
                         rowidx.reshape(_NW * _ROWCAP))

    gathered = pl.pallas_call(
        _label_kernel,
        grid=(_R // _CHL,),
        in_specs=[
            pl.BlockSpec((_CHL, _C, _S), lambda i: (i, 0, 0)),
            pl.BlockSpec((_CHL, 1), lambda i: (i, 0)),
        ],
        out_specs=pl.BlockSpec((_CHL, _C), lambda i: (i, 0)),
        out_shape=jax.ShapeDtypeStruct((_R, _C), jnp.float32),
    )(labels_t, gold_heads.reshape(_R, 1))

    out = pl.pallas_call(
        _final_kernel,
        out_shape=jax.ShapeDtypeStruct((1, 1), jnp.float32),
    )(compact.reshape(1, _TOTCAP), pos.reshape(_P, 1), pos.reshape(1, _P),
      tstats, gathered, gold2)

    return out[0, 0]


# speedup vs baseline: 16.4695x; 1.1182x over previous
import functools

import jax
import jax.numpy as jnp
import numpy as np
from jax import lax
from jax.experimental import pallas as pl
from jax.experimental.pallas import tpu as pltpu
from jax.experimental.pallas import tpu_sc as plsc

_B, _S, _C = 4, 512, 48
_R = _B * _S
_P = _R
_NNEG = 5 * _P
_MARGIN = 0.3
_ALPHA = 0.5
_SMOOTH = 0.1
_BASE = _SMOOTH / (_C - 1)

_NW = 32
_GRP = _R // _NW
_CHUNK = _GRP * _S
_CAP = 12288
_PAD = 128
_TOTCAP = _CAP + _NW * _PAD
_ROWCAP = _CAP // _PAD
_PAD_SHIFT = _PAD.bit_length() - 1

_INT_MIN = np.int32(-2147483648)
_NEG_SENTINEL = -1.0e9


def _sort_key(i):
    return i ^ (lax.shift_right_arithmetic(i, 31) & 0x7FFFFFFF)


def _f32_key(x):
    return _sort_key(lax.bitcast_convert_type(x, jnp.int32))


def _stats_kernel(edge_ref, heads_ref, pos_ref, rowidx_ref, taub_ref,
                  tstats_ref):
    edge = edge_ref[...]
    heads = heads_ref[...]

    col = lax.broadcasted_iota(jnp.int32, (_NW, _GRP, _S), 2)
    sel = col == heads[:, :, None]
    pos = jnp.sum(jnp.where(sel, edge, 0.0), axis=2)

    keys = _f32_key(edge)
    pkeys = _f32_key(pos)

    def count_ge(t):
        return (jnp.sum((keys >= t).astype(jnp.int32))
                - jnp.sum((pkeys >= t).astype(jnp.int32)))

    t0 = jnp.where(count_ge(jnp.int32(0)) >= _NNEG, jnp.int32(0), _INT_MIN)

    def bs_body(k, t):
        trial = t + lax.shift_left(jnp.int32(1), jnp.int32(30) - k)
        return jnp.where(count_ge(trial) >= _NNEG, trial, t)

    tau = lax.fori_loop(0, 31, bs_body, t0)

    cnt_gt = (jnp.sum((keys > tau).astype(jnp.int32))
              - jnp.sum((pkeys > tau).astype(jnp.int32)))

    gt = (keys > tau).astype(jnp.int32)
    counts = jnp.sum(jnp.sum(gt, axis=2), axis=1, keepdims=True)
    rc = (counts + (_PAD - 1)) & (-_PAD)
    total = jnp.sum(rc)
    row = lax.broadcasted_iota(jnp.int32, (_NW, 1), 0)

    def off_body(wi, acc):
        pre = jnp.sum(jnp.where(row < wi, rc, 0))
        return jnp.where(row == wi, pre, acc)

    offs = lax.fori_loop(0, _NW, off_body, jnp.zeros((_NW, 1), jnp.int32))

    pos_ref[...] = pos
    colr = lax.broadcasted_iota(jnp.int32, (_NW, _ROWCAP), 1)
    rowidx_ref[...] = jnp.where(
        colr < lax.shift_right_logical(rc, _PAD_SHIFT),
        lax.shift_right_logical(offs, _PAD_SHIFT) + colr, -1)
    taub_ref[...] = jnp.full((1, 16), 0, jnp.int32) + tau
    lane = lax.broadcasted_iota(jnp.int32, (1, 128), 1)
    tstats_ref[...] = jnp.where(
        lane == 0, tau,
        jnp.where(lane == 1, cnt_gt, jnp.where(lane == 2, total, 0)))


def _sc_filter_body(edge_ref, tau_ref, staging_ref, vchunk, cbuf, tbuf):
    w = lax.axis_index("s") * 2 + lax.axis_index("c")

    pltpu.sync_copy(tau_ref, tbuf)
    tau_v = tbuf[...]

    pltpu.sync_copy(edge_ref.at[pl.ds(w * _CHUNK, _CHUNK)], vchunk)

    def fill_body(j, _):
        cbuf[pl.ds(j * 16, 16)] = jnp.full((16,), _NEG_SENTINEL, jnp.float32)
        return 0

    lax.fori_loop(0, _CAP // 16, fill_body, 0)

    def filt_body(j, wp):
        v = vchunk[pl.ds(j * 16, 16)]
        key = _sort_key(plsc.bitcast(v, jnp.int32))
        m = key > tau_v
        mi = m.astype(jnp.int32)
        dst = wp + plsc.cumsum(mi) - 1
        plsc.store_scatter(cbuf, [dst], v, mask=m)
        return wp + jnp.sum(mi)

    lax.fori_loop(0, _CHUNK // 16, filt_body, jnp.int32(0))

    pltpu.sync_copy(cbuf, staging_ref.at[pl.ds(w * _CAP, _CAP)])


def _sc_pack_body(staging_ref, rowidx_ref, compact_ref, sbuf, ridx, sem2):
    w = lax.axis_index("s") * 2 + lax.axis_index("c")

    pltpu.sync_copy(staging_ref.at[pl.ds(w * _ROWCAP, _ROWCAP)], sbuf)
    pltpu.sync_copy(rowidx_ref.at[pl.ds(w * _ROWCAP, _ROWCAP)], ridx)
    pltpu.async_copy(
        sbuf, compact_ref.at[plsc.Indices(ridx, ignored_value=-1)],
        sem2).wait()


def _sc_mesh():
    return plsc.VectorSubcoreMesh(core_axis_name="c", subcore_axis_name="s",
                                  num_cores=2, num_subcores=16)


@functools.cache
def _sc_filter():
    return pl.kernel(
        _sc_filter_body,
        out_type=jax.ShapeDtypeStruct((_NW * _CAP,), jnp.float32),
        mesh=_sc_mesh(),
        compiler_params=pltpu.CompilerParams(use_tc_tiling_on_sc=False,
                                             needs_layout_passes=False),
        scratch_types=[
            pltpu.VMEM((_CHUNK,), jnp.float32),
            pltpu.VMEM((_CAP,), jnp.float32),
            pltpu.VMEM((16,), jnp.int32),
        ],
    )


@functools.cache
def _sc_pack():
    return pl.kernel(
        _sc_pack_body,
        out_type=jax.ShapeDtypeStruct((_TOTCAP // _PAD, _PAD), jnp.float32),
        mesh=_sc_mesh(),
        compiler_params=pltpu.CompilerParams(use_tc_tiling_on_sc=False),
        scratch_types=[
            pltpu.VMEM((_ROWCAP, _PAD), jnp.float32),
            pltpu.VMEM((_ROWCAP,), jnp.int32),
            pltpu.SemaphoreType.DMA,
        ],
    )


_CHL = 32


def _label_kernel(labels_ref, heads_ref, gathered_ref):
    x3 = labels_ref[...]
    h = heads_ref[...]
    hsel = lax.broadcasted_iota(jnp.int32, (_CHL, _C, _S), 2) == h[:, :, None]
    gathered_ref[...] = jnp.sum(jnp.where(hsel, x3, 0.0), axis=2)


def _final_kernel(crow_ref, pos_col_ref, pos_row_ref, tstats_ref,
                  gathered_ref, gold_ref, out_ref):
    lane = lax.broadcasted_iota(jnp.int32, (1, 128), 1)
    tstats = tstats_ref[...]
    tau = jnp.sum(jnp.where(lane == 0, tstats, 0))
    cnt_gt = jnp.sum(jnp.where(lane == 1, tstats, 0))
    total = jnp.sum(jnp.where(lane == 2, tstats, 0))
    ties = (_NNEG - cnt_gt).astype(jnp.float32)
    tau_f = lax.bitcast_convert_type(_sort_key(tau), jnp.float32)

    pos_col = pos_col_ref[...]
    pos_row = pos_row_ref[...]

    def pair_body(k, acc_row):
        c = crow_ref[:, pl.ds(k * 128, 128)]
        gidx = k * 128 + lax.broadcasted_iota(jnp.int32, (1, 128), 1)
        c = jnp.where(gidx < total, c, _NEG_SENTINEL)
        part = jnp.sum(jnp.maximum(c + (_MARGIN - pos_col), 0.0),
                       axis=0, keepdims=True)
        return acc_row + part

    acc_row = lax.fori_loop(0, _TOTCAP // 128, pair_body,
                            jnp.zeros((1, 128), jnp.float32))
    acc = jnp.sum(acc_row)

    gp_row = jnp.where(_f32_key(pos_row) > tau, pos_row, _NEG_SENTINEL)
    acc = acc - jnp.sum(jnp.maximum(gp_row + (_MARGIN - pos_col), 0.0))
    acc = acc + ties * jnp.sum(jnp.maximum(tau_f + (_MARGIN - pos_col), 0.0))

    margin_loss = acc / jnp.float32(_P * _NNEG)

    g = gathered_ref[...]
    tgt = gold_ref[...]
    m = jnp.max(g, axis=1, keepdims=True)
    lse = m + jnp.log(jnp.sum(jnp.exp(g - m), axis=1, keepdims=True))
    sum_logp = jnp.sum(g, axis=1, keepdims=True) - _C * lse
    clane = lax.broadcasted_iota(jnp.int32, (_R, _C), 1)
    xt = jnp.sum(jnp.where(clane == tgt, g, 0.0), axis=1, keepdims=True)
    logp_t = xt - lse
    row_loss = -(_BASE * sum_logp) - (1.0 - _SMOOTH - _BASE) * logp_t
    label_loss = jnp.sum(row_loss) / jnp.float32(_R)

    out_ref[...] = jnp.full((1, 1),
                            _ALPHA * margin_loss + (1.0 - _ALPHA) * label_loss,
                            jnp.float32)


@jax.jit
def kernel(edge_scores, label_scores, gold_heads, gold_labels):
    edge3 = edge_scores.reshape(_NW, _GRP, _S)
    heads2 = gold_heads.reshape(_NW, _GRP)
    gold2 = gold_labels.reshape(_R, 1)
    labels_t = label_scores.transpose(0, 1, 3, 2).reshape(_R, _C, _S)
    edge_flat = edge_scores.reshape(_R * _S)

    pos, rowidx, taub, tstats = pl.pallas_call(
        _stats_kernel,
        out_shape=(
            jax.ShapeDtypeStruct((_NW, _GRP), jnp.float32),
            jax.ShapeDtypeStruct((_NW, _ROWCAP), jnp.int32),
            jax.ShapeDtypeStruct((1, 16), jnp.int32),
            jax.ShapeDtypeStruct((1, 128), jnp.int32),
        ),
    )(edge3, heads2)

    staging = _sc_filter()(edge_flat, taub.reshape(16))

    gathered = pl.pallas_call(
        _label_kernel,
        grid=(_R // _CHL,),
        in_specs=[
            pl.BlockSpec((_CHL, _C, _S), lambda i: (i, 0, 0)),
            pl.BlockSpec((_CHL, 1), lambda i: (i, 0)),
        ],
        out_specs=pl.BlockSpec((_CHL, _C), lambda i: (i, 0)),
        out_shape=jax.ShapeDtypeStruct((_R, _C), jnp.float32),
    )(labels_t, gold_heads.reshape(_R, 1))

    compact = _sc_pack()(staging.reshape(_NW * _ROWCAP, _PAD),
                         rowidx.reshape(_NW * _ROWCAP))

    out = pl.pallas_call(
        _final_kernel,
        out_shape=jax.ShapeDtypeStruct((1, 1), jnp.float32),
    )(compact.reshape(1, _TOTCAP), pos.reshape(_P, 1), pos.reshape(1, _P),
      tstats, gathered, gold2)

    return out[0, 0]


# speedup vs baseline: 17.4838x; 1.0616x over previous
import functools

import jax
import jax.numpy as jnp
import numpy as np
from jax import lax
from jax.experimental import pallas as pl
from jax.experimental.pallas import tpu as pltpu
from jax.experimental.pallas import tpu_sc as plsc

_B, _S, _C = 4, 512, 48
_R = _B * _S
_P = _R
_NNEG = 5 * _P
_MARGIN = 0.3
_ALPHA = 0.5
_SMOOTH = 0.1
_BASE = _SMOOTH / (_C - 1)

_NW = 32
_GRP = _R // _NW
_CHUNK = _GRP * _S
_CAP = 12288
_PAD = 128
_TOTCAP = _CAP + _NW * _PAD
_ROWCAP = _CAP // _PAD
_PAD_SHIFT = _PAD.bit_length() - 1

_INT_MIN = np.int32(-2147483648)
_NEG_SENTINEL = -1.0e9


def _sort_key(i):
    return i ^ (lax.shift_right_arithmetic(i, 31) & 0x7FFFFFFF)


def _f32_key(x):
    return _sort_key(lax.bitcast_convert_type(x, jnp.int32))


def _stats_kernel(edge_ref, heads_ref, pos_ref, rowidx_ref, taub_ref,
                  tstats_ref):
    edge = edge_ref[...]
    heads = heads_ref[...]

    col = lax.broadcasted_iota(jnp.int32, (_NW, _GRP, _S), 2)
    sel = col == heads[:, :, None]
    pos = jnp.sum(jnp.where(sel, edge, 0.0), axis=2)

    keys = _f32_key(edge)
    pkeys = _f32_key(pos)

    def count_ge(t):
        return (jnp.sum((keys >= t).astype(jnp.int32))
                - jnp.sum((pkeys >= t).astype(jnp.int32)))

    t0 = jnp.where(count_ge(jnp.int32(0)) >= _NNEG, jnp.int32(0), _INT_MIN)

    def bs_body(k, t):
        trial = t + lax.shift_left(jnp.int32(1), jnp.int32(30) - k)
        return jnp.where(count_ge(trial) >= _NNEG, trial, t)

    tau = lax.fori_loop(0, 31, bs_body, t0)

    cnt_gt = (jnp.sum((keys > tau).astype(jnp.int32))
              - jnp.sum((pkeys > tau).astype(jnp.int32)))

    gt = (keys > tau).astype(jnp.int32)
    counts = jnp.sum(jnp.sum(gt, axis=2), axis=1, keepdims=True)
    rc = (counts + (_PAD - 1)) & (-_PAD)
    total = jnp.sum(rc)
    row = lax.broadcasted_iota(jnp.int32, (_NW, 1), 0)

    def off_body(wi, acc):
        pre = jnp.sum(jnp.where(row < wi, rc, 0))
        return jnp.where(row == wi, pre, acc)

    offs = lax.fori_loop(0, _NW, off_body, jnp.zeros((_NW, 1), jnp.int32))

    pos_ref[...] = pos
    colr = lax.broadcasted_iota(jnp.int32, (_NW, _ROWCAP), 1)
    rowidx_ref[...] = jnp.where(
        colr < lax.shift_right_logical(rc, _PAD_SHIFT),
        lax.shift_right_logical(offs, _PAD_SHIFT) + colr, -1)
    taub_ref[...] = jnp.full((1, 16), 0, jnp.int32) + tau
    lane = lax.broadcasted_iota(jnp.int32, (1, 128), 1)
    tstats_ref[...] = jnp.where(
        lane == 0, tau,
        jnp.where(lane == 1, cnt_gt, jnp.where(lane == 2, total, 0)))


def _sc_filter_body(edge_ref, tau_ref, staging_ref, vchunk, cbuf, tbuf):
    w = lax.axis_index("s") * 2 + lax.axis_index("c")

    pltpu.sync_copy(tau_ref, tbuf)
    tau_v = tbuf[...]

    pltpu.sync_copy(edge_ref.at[pl.ds(w * _CHUNK, _CHUNK)], vchunk)

    def fill_body(j, _):
        cbuf[pl.ds(j * 16, 16)] = jnp.full((16,), _NEG_SENTINEL, jnp.float32)
        return 0

    lax.fori_loop(0, _CAP // 16, fill_body, 0)

    def filt_body(j, wp):
        v1 = vchunk[pl.ds(j * 32, 16)]
        v2 = vchunk[pl.ds(j * 32 + 16, 16)]
        k1 = _sort_key(plsc.bitcast(v1, jnp.int32))
        k2 = _sort_key(plsc.bitcast(v2, jnp.int32))
        m1 = k1 > tau_v
        m2 = k2 > tau_v
        mi1 = m1.astype(jnp.int32)
        mi2 = m2.astype(jnp.int32)
        plsc.store_scatter(cbuf, [wp + plsc.cumsum(mi1) - 1], v1, mask=m1)
        wp1 = wp + jnp.sum(mi1)
        plsc.store_scatter(cbuf, [wp1 + plsc.cumsum(mi2) - 1], v2, mask=m2)
        return wp1 + jnp.sum(mi2)

    lax.fori_loop(0, _CHUNK // 32, filt_body, jnp.int32(0))

    pltpu.sync_copy(cbuf, staging_ref.at[pl.ds(w * _CAP, _CAP)])


def _sc_pack_body(staging_ref, rowidx_ref, compact_ref, sbuf, ridx, sem2):
    w = lax.axis_index("s") * 2 + lax.axis_index("c")

    pltpu.sync_copy(staging_ref.at[pl.ds(w * _ROWCAP, _ROWCAP)], sbuf)
    pltpu.sync_copy(rowidx_ref.at[pl.ds(w * _ROWCAP, _ROWCAP)], ridx)
    pltpu.async_copy(
        sbuf, compact_ref.at[plsc.Indices(ridx, ignored_value=-1)],
        sem2).wait()


def _sc_mesh():
    return plsc.VectorSubcoreMesh(core_axis_name="c", subcore_axis_name="s",
                                  num_cores=2, num_subcores=16)


@functools.cache
def _sc_filter():
    return pl.kernel(
        _sc_filter_body,
        out_type=jax.ShapeDtypeStruct((_NW * _CAP,), jnp.float32),
        mesh=_sc_mesh(),
        compiler_params=pltpu.CompilerParams(use_tc_tiling_on_sc=False,
                                             needs_layout_passes=False),
        scratch_types=[
            pltpu.VMEM((_CHUNK,), jnp.float32),
            pltpu.VMEM((_CAP,), jnp.float32),
            pltpu.VMEM((16,), jnp.int32),
        ],
    )


@functools.cache
def _sc_pack():
    return pl.kernel(
        _sc_pack_body,
        out_type=jax.ShapeDtypeStruct((_TOTCAP // _PAD, _PAD), jnp.float32),
        mesh=_sc_mesh(),
        compiler_params=pltpu.CompilerParams(use_tc_tiling_on_sc=False),
        scratch_types=[
            pltpu.VMEM((_ROWCAP, _PAD), jnp.float32),
            pltpu.VMEM((_ROWCAP,), jnp.int32),
            pltpu.SemaphoreType.DMA,
        ],
    )


_CHL = 32


def _label_kernel(labels_ref, heads_ref, gathered_ref):
    x3 = labels_ref[...]
    h = heads_ref[...]
    hsel = lax.broadcasted_iota(jnp.int32, (_CHL, 1, _S), 2) == h[:, :, None]
    gathered_ref[...] = jnp.sum(jnp.where(hsel, x3, 0.0), axis=2)


def _final_kernel(crow_ref, pos_col_ref, pos_row_ref, tstats_ref,
                  gathered_ref, gold_ref, out_ref):
    lane = lax.broadcasted_iota(jnp.int32, (1, 128), 1)
    tstats = tstats_ref[...]
    tau = jnp.sum(jnp.where(lane == 0, tstats, 0))
    cnt_gt = jnp.sum(jnp.where(lane == 1, tstats, 0))
    total = jnp.sum(jnp.where(lane == 2, tstats, 0))
    ties = (_NNEG - cnt_gt).astype(jnp.float32)
    tau_f = lax.bitcast_convert_type(_sort_key(tau), jnp.float32)

    pos_col = pos_col_ref[...]
    pos_row = pos_row_ref[...]

    def pair_body(k, acc_row):
        c = crow_ref[:, pl.ds(k * 128, 128)]
        gidx = k * 128 + lax.broadcasted_iota(jnp.int32, (1, 128), 1)
        c = jnp.where(gidx < total, c, _NEG_SENTINEL)
        part = jnp.sum(jnp.maximum(c + (_MARGIN - pos_col), 0.0),
                       axis=0, keepdims=True)
        return acc_row + part

    nchunks = lax.shift_right_logical(total + 127, 7)
    acc_row = lax.fori_loop(0, nchunks, pair_body,
                            jnp.zeros((1, 128), jnp.float32))
    acc = jnp.sum(acc_row)

    gp_row = jnp.where(_f32_key(pos_row) > tau, pos_row, _NEG_SENTINEL)
    acc = acc - jnp.sum(jnp.maximum(gp_row + (_MARGIN - pos_col), 0.0))
    acc = acc + ties * jnp.sum(jnp.maximum(tau_f + (_MARGIN - pos_col), 0.0))

    margin_loss = acc / jnp.float32(_P * _NNEG)

    g = gathered_ref[...]
    tgt = gold_ref[...]
    m = jnp.max(g, axis=1, keepdims=True)
    lse = m + jnp.log(jnp.sum(jnp.exp(g - m), axis=1, keepdims=True))
    sum_logp = jnp.sum(g, axis=1, keepdims=True) - _C * lse
    clane = lax.broadcasted_iota(jnp.int32, (_R, _C), 1)
    xt = jnp.sum(jnp.where(clane == tgt, g, 0.0), axis=1, keepdims=True)
    logp_t = xt - lse
    row_loss = -(_BASE * sum_logp) - (1.0 - _SMOOTH - _BASE) * logp_t
    label_loss = jnp.sum(row_loss) / jnp.float32(_R)

    out_ref[...] = jnp.full((1, 1),
                            _ALPHA * margin_loss + (1.0 - _ALPHA) * label_loss,
                            jnp.float32)


@jax.jit
def kernel(edge_scores, label_scores, gold_heads, gold_labels):
    edge3 = edge_scores.reshape(_NW, _GRP, _S)
    heads2 = gold_heads.reshape(_NW, _GRP)
    gold2 = gold_labels.reshape(_R, 1)
    labels_t = label_scores.transpose(0, 1, 3, 2).reshape(_R, _C, _S)
    edge_flat = edge_scores.reshape(_R * _S)

    pos, rowidx, taub, tstats = pl.pallas_call(
        _stats_kernel,
        out_shape=(
            jax.ShapeDtypeStruct((_NW, _GRP), jnp.float32),
            jax.ShapeDtypeStruct((_NW, _ROWCAP), jnp.int32),
            jax.ShapeDtypeStruct((1, 16), jnp.int32),
            jax.ShapeDtypeStruct((1, 128), jnp.int32),
        ),
    )(edge3, heads2)

    staging = _sc_filter()(edge_flat, taub.reshape(16))

    gathered = pl.pallas_call(
        _label_kernel,
        grid=(_R // _CHL,),
        in_specs=[
            pl.BlockSpec((_CHL, _C, _S), lambda i: (i, 0, 0)),
            pl.BlockSpec((_CHL, 1), lambda i: (i, 0)),
        ],
        out_specs=pl.BlockSpec((_CHL, _C), lambda i: (i, 0)),
        out_shape=jax.ShapeDtypeStruct((_R, _C), jnp.float32),
    )(labels_t, gold_heads.reshape(_R, 1))

    compact = _sc_pack()(staging.reshape(_NW * _ROWCAP, _PAD),
                         rowidx.reshape(_NW * _ROWCAP))

    out = pl.pallas_call(
        _final_kernel,
        out_shape=jax.ShapeDtypeStruct((1, 1), jnp.float32),
    )(compact.reshape(1, _TOTCAP), pos.reshape(_P, 1), pos.reshape(1, _P),
      tstats, gathered, gold2)

    return out[0, 0]
